# baseline (speedup 1.0000x reference)
import jax
import jax.numpy as jnp
from jax import lax
from jax.experimental import pallas as pl
from jax.experimental.pallas import tpu as pltpu
from jax.experimental.pallas import tpu_sc as plsc

N = 10000
E = 320000
F = 128

NC = 2
NS = 16
TILES = NC * NS
ET = E // TILES
CHUNK = 80
NCHUNK = ET // CHUNK
IB = 25
NSLAB = NCHUNK // IB
NBUF = 4
N_PAD = 10112
ROWS_PT = N_PAD // NS

_mesh = plsc.VectorSubcoreMesh(core_axis_name="c", subcore_axis_name="s")


def _seg_sum_body(h_hbm, src_hbm, dst_hbm, zeros_hbm, out_hbm,
                  idx_s, idx_d, b0, b1, b2, b3, acc_sh,
                  g0s, g1s, g2s, g3s, s0s, s1s, s2s, s3s):
    c = lax.axis_index("c")
    s = lax.axis_index("s")
    wid = s * NC + c
    row0 = s * ROWS_PT
    bufs = (b0, b1, b2, b3)
    gsems = (g0s, g1s, g2s, g3s)
    ssems = (s0s, s1s, s2s, s3s)

    pltpu.sync_copy(zeros_hbm.at[pl.ds(row0, ROWS_PT)],
                    acc_sh.at[pl.ds(row0, ROWS_PT)])
    plsc.subcore_barrier()

    def gather(g, b):
        pltpu.async_copy(h_hbm.at[idx_s.at[g]], bufs[b], gsems[b])

    def gather_wait(b):
        pltpu.make_async_copy(h_hbm.at[idx_s.at[0]], bufs[b], gsems[b]).wait()

    def scat(g, b):
        pltpu.async_copy(bufs[b], acc_sh.at[idx_d.at[g]], ssems[b], add=True)

    def scat_wait(b):
        pltpu.make_async_copy(bufs[b], acc_sh.at[idx_d.at[0]], ssems[b]).wait()

    def slab(sl, carry):
        pltpu.sync_copy(src_hbm.at[wid].at[sl], idx_s)
        pltpu.sync_copy(dst_hbm.at[wid].at[sl], idx_d)
        for j in range(NBUF):
            gather(j, j)
        for j in range(IB):
            b = j % NBUF
            gather_wait(b)
            scat(j, b)
            if j >= 2:
                scat_wait((j - 2) % NBUF)
                if j + 2 < IB:
                    gather(j + 2, (j + 2) % NBUF)
        scat_wait((IB - 2) % NBUF)
        scat_wait((IB - 1) % NBUF)
        return carry

    lax.fori_loop(0, NSLAB, slab, 0)
    plsc.subcore_barrier()

    pltpu.sync_copy(acc_sh.at[pl.ds(row0, ROWS_PT)],
                    out_hbm.at[pl.ds(c * N_PAD + row0, ROWS_PT)])


_seg_sum = pl.kernel(
    _seg_sum_body,
    out_type=jax.ShapeDtypeStruct((2 * N_PAD, F), jnp.float32),
    mesh=_mesh,
    scratch_types=[
        pltpu.VMEM((IB, CHUNK), jnp.int32),
        pltpu.VMEM((IB, CHUNK), jnp.int32),
        pltpu.VMEM((CHUNK, F), jnp.float32),
        pltpu.VMEM((CHUNK, F), jnp.float32),
        pltpu.VMEM((CHUNK, F), jnp.float32),
        pltpu.VMEM((CHUNK, F), jnp.float32),
        pltpu.VMEM_SHARED((N_PAD, F), jnp.float32),
        pltpu.SemaphoreType.DMA,
        pltpu.SemaphoreType.DMA,
        pltpu.SemaphoreType.DMA,
        pltpu.SemaphoreType.DMA,
        pltpu.SemaphoreType.DMA,
        pltpu.SemaphoreType.DMA,
        pltpu.SemaphoreType.DMA,
        pltpu.SemaphoreType.DMA,
    ],
)


def _score_body(a_hbm, b_hbm, src_hbm, dst_hbm, out_hbm,
                a_v, b_v, src_v, dst_v, out_v):
    c = lax.axis_index("c")
    s = lax.axis_index("s")
    wid = s * NC + c
    base = wid * ET

    pltpu.sync_copy(a_hbm, a_v)
    pltpu.sync_copy(b_hbm, b_v)
    pltpu.sync_copy(src_hbm.at[pl.ds(base, ET)], src_v)
    pltpu.sync_copy(dst_hbm.at[pl.ds(base, ET)], dst_v)

    def step(i, carry):
        for k in range(5):
            o = (i * 5 + k) * 16
            isrc = src_v[pl.ds(o, 16)]
            idst = dst_v[pl.ds(o, 16)]
            va = plsc.load_gather(a_v, [isrc])
            vb = plsc.load_gather(b_v, [idst])
            z = va + vb
            out_v[pl.ds(o, 16)] = 1.0 / (1.0 + jnp.exp(-z))
        return carry

    lax.fori_loop(0, ET // 80, step, 0)
    pltpu.sync_copy(out_v, out_hbm.at[pl.ds(base, ET)])


_score = pl.kernel(
    _score_body,
    out_type=jax.ShapeDtypeStruct((E,), jnp.float32),
    mesh=_mesh,
    compiler_params=pltpu.CompilerParams(needs_layout_passes=False),
    scratch_types=[
        pltpu.VMEM((N,), jnp.float32),
        pltpu.VMEM((N,), jnp.float32),
        pltpu.VMEM((ET,), jnp.int32),
        pltpu.VMEM((ET,), jnp.int32),
        pltpu.VMEM((ET,), jnp.float32),
    ],
)


def _bn_fast(t, g, b):
    s1 = jnp.sum(t, axis=0)
    s2 = jnp.sum(t * t, axis=0)
    m = s1 * (1.0 / N)
    v = s2 * (1.0 / N) - m * m
    return (t - m) * lax.rsqrt(v + 1e-5) * g + b


def _gin_compute(h, agg, W1, b1, gm, bm, W2, b2, eps, go, bo):
    t = (1.0 + eps) * h + agg
    t = jnp.dot(t, W1, preferred_element_type=jnp.float32) + b1
    t = jnp.maximum(_bn_fast(t, gm, bm), 0.0)
    t = jnp.dot(t, W2, preferred_element_type=jnp.float32) + b2
    return _bn_fast(t, go, bo)


def _gin0_body(h_ref, p_ref, W1_ref, b1_ref, gm_ref, bm_ref, W2_ref, b2_ref,
               eps_ref, go_ref, bo_ref, out_ref):
    p = p_ref[...]
    agg = p[:N] + p[N_PAD:N_PAD + N]
    t = _gin_compute(h_ref[...], agg, W1_ref[...], b1_ref[...], gm_ref[...],
                     bm_ref[...], W2_ref[...], b2_ref[...], eps_ref[0, 0],
                     go_ref[...], bo_ref[...])
    out_ref[...] = jnp.maximum(t, 0.0)


def _gin1_body(h_ref, p_ref, W1_ref, b1_ref, gm_ref, bm_ref, W2_ref, b2_ref,
               eps_ref, go_ref, bo_ref, attW_ref, attb_ref, ab_ref):
    p = p_ref[...]
    agg = p[:N] + p[N_PAD:N_PAD + N]
    t = _gin_compute(h_ref[...], agg, W1_ref[...], b1_ref[...], gm_ref[...],
                     bm_ref[...], W2_ref[...], b2_ref[...], eps_ref[0, 0],
                     go_ref[...], bo_ref[...])
    attW = attW_ref[...]
    a = jnp.dot(t, attW[:F], preferred_element_type=jnp.float32) + attb_ref[0, 0]
    b = jnp.dot(t, attW[F:], preferred_element_type=jnp.float32)
    ab_ref[...] = jnp.concatenate([a, b], axis=1)


_gin0 = pl.pallas_call(
    _gin0_body,
    out_shape=jax.ShapeDtypeStruct((N, F), jnp.float32),
)

_gin1 = pl.pallas_call(
    _gin1_body,
    out_shape=jax.ShapeDtypeStruct((N, 2), jnp.float32),
)


def kernel(x, edge_index, p0_W1, p0_b1, p0_gm, p0_bm, p0_W2, p0_b2, p0_eps,
           p0_go, p0_bo, p1_W1, p1_b1, p1_gm, p1_bm, p1_W2, p1_b2, p1_eps,
           p1_go, p1_bo, att_W, att_b):
    src = edge_index[0]
    dst = edge_index[1]
    src3 = src.reshape(TILES, NSLAB, IB, CHUNK)
    dst3 = dst.reshape(TILES, NSLAB, IB, CHUNK)
    zeros = jnp.zeros((N_PAD, F), jnp.float32)

    def r2(v):
        return v.reshape(1, -1)

    part0 = _seg_sum(x, src3, dst3, zeros)
    h = _gin0(x, part0, p0_W1, r2(p0_b1), r2(p0_gm), r2(p0_bm), p0_W2,
              r2(p0_b2), r2(p0_eps), r2(p0_go), r2(p0_bo))
    part1 = _seg_sum(h, src3, dst3, zeros)
    ab = _gin1(h, part1, p1_W1, r2(p1_b1), r2(p1_gm), r2(p1_bm), p1_W2,
               r2(p1_b2), r2(p1_eps), r2(p1_go), r2(p1_bo), att_W, r2(att_b))
    return _score(ab[:, 0], ab[:, 1], src, dst)


# speedup vs baseline: 1.1146x; 1.1146x over previous
import jax
import jax.numpy as jnp
from jax import lax
from jax.experimental import pallas as pl
from jax.experimental.pallas import tpu as pltpu
from jax.experimental.pallas import tpu_sc as plsc

N = 10000
E = 320000
F = 128

NC = 2
NS = 16
TILES = NC * NS
ET = E // TILES
CHUNK = 100
NCHUNK = ET // CHUNK
IB = 25
NSLAB = NCHUNK // IB
NBUF = 3
N_PAD = 10112
ROWS_PT = N_PAD // NS

_mesh = plsc.VectorSubcoreMesh(core_axis_name="c", subcore_axis_name="s")


def _seg_sum_body(h_hbm, src_hbm, dst_hbm, zeros_hbm, out_hbm,
                  idx_s, idx_d, b0, b1, b2, acc_sh,
                  g0s, g1s, g2s, s0s, s1s, s2s):
    c = lax.axis_index("c")
    s = lax.axis_index("s")
    wid = s * NC + c
    row0 = s * ROWS_PT
    bufs = (b0, b1, b2)
    gsems = (g0s, g1s, g2s)
    ssems = (s0s, s1s, s2s)

    pltpu.sync_copy(zeros_hbm.at[pl.ds(row0, ROWS_PT)],
                    acc_sh.at[pl.ds(row0, ROWS_PT)])
    plsc.subcore_barrier()

    def gather(g, b):
        pltpu.async_copy(h_hbm.at[idx_s.at[g]], bufs[b], gsems[b])

    def gather_wait(b):
        pltpu.make_async_copy(h_hbm.at[idx_s.at[0]], bufs[b], gsems[b]).wait()

    def scat(g, b):
        pltpu.async_copy(bufs[b], acc_sh.at[idx_d.at[g]], ssems[b], add=True)

    def scat_wait(b):
        pltpu.make_async_copy(bufs[b], acc_sh.at[idx_d.at[0]], ssems[b]).wait()

    def slab(sl, carry):
        pltpu.sync_copy(src_hbm.at[wid].at[sl], idx_s)
        pltpu.sync_copy(dst_hbm.at[wid].at[sl], idx_d)
        for j in range(2):
            gather(j, j)
        for j in range(IB):
            b = j % NBUF
            gather_wait(b)
            scat(j, b)
            if j >= 1:
                scat_wait((j - 1) % NBUF)
            if j + 2 < IB:
                gather(j + 2, (j + 2) % NBUF)
        scat_wait((IB - 1) % NBUF)
        return carry

    lax.fori_loop(0, NSLAB, slab, 0)
    plsc.subcore_barrier()

    pltpu.sync_copy(acc_sh.at[pl.ds(row0, ROWS_PT)],
                    out_hbm.at[pl.ds(c * N_PAD + row0, ROWS_PT)])


_seg_sum = pl.kernel(
    _seg_sum_body,
    out_type=jax.ShapeDtypeStruct((2 * N_PAD, F), jnp.float32),
    mesh=_mesh,
    scratch_types=[
        pltpu.VMEM((IB, CHUNK), jnp.int32),
        pltpu.VMEM((IB, CHUNK), jnp.int32),
        pltpu.VMEM((CHUNK, F), jnp.float32),
        pltpu.VMEM((CHUNK, F), jnp.float32),
        pltpu.VMEM((CHUNK, F), jnp.float32),
        pltpu.VMEM_SHARED((N_PAD, F), jnp.float32),
        pltpu.SemaphoreType.DMA,
        pltpu.SemaphoreType.DMA,
        pltpu.SemaphoreType.DMA,
        pltpu.SemaphoreType.DMA,
        pltpu.SemaphoreType.DMA,
        pltpu.SemaphoreType.DMA,
    ],
)


def _score_body(a_hbm, b_hbm, src_hbm, dst_hbm, out_hbm,
                a_v, b_v, src_v, dst_v, out_v):
    c = lax.axis_index("c")
    s = lax.axis_index("s")
    wid = s * NC + c
    base = wid * ET

    pltpu.sync_copy(a_hbm, a_v)
    pltpu.sync_copy(b_hbm, b_v)
    pltpu.sync_copy(src_hbm.at[pl.ds(base, ET)], src_v)
    pltpu.sync_copy(dst_hbm.at[pl.ds(base, ET)], dst_v)

    def step(i, carry):
        for k in range(5):
            o = (i * 5 + k) * 16
            isrc = src_v[pl.ds(o, 16)]
            idst = dst_v[pl.ds(o, 16)]
            va = plsc.load_gather(a_v, [isrc])
            vb = plsc.load_gather(b_v, [idst])
            z = va + vb
            out_v[pl.ds(o, 16)] = 1.0 / (1.0 + jnp.exp(-z))
        return carry

    lax.fori_loop(0, ET // 80, step, 0)
    pltpu.sync_copy(out_v, out_hbm.at[pl.ds(base, ET)])


_score = pl.kernel(
    _score_body,
    out_type=jax.ShapeDtypeStruct((E,), jnp.float32),
    mesh=_mesh,
    compiler_params=pltpu.CompilerParams(needs_layout_passes=False),
    scratch_types=[
        pltpu.VMEM((N,), jnp.float32),
        pltpu.VMEM((N,), jnp.float32),
        pltpu.VMEM((ET,), jnp.int32),
        pltpu.VMEM((ET,), jnp.int32),
        pltpu.VMEM((ET,), jnp.float32),
    ],
)


def _bn_fast(t, g, b):
    s1 = jnp.sum(t, axis=0)
    s2 = jnp.sum(t * t, axis=0)
    m = s1 * (1.0 / N)
    v = s2 * (1.0 / N) - m * m
    return (t - m) * lax.rsqrt(v + 1e-5) * g + b


def _gin_compute(h, agg, W1, b1, gm, bm, W2, b2, eps, go, bo):
    t = (1.0 + eps) * h + agg
    t = jnp.dot(t, W1, preferred_element_type=jnp.float32) + b1
    t = jnp.maximum(_bn_fast(t, gm, bm), 0.0)
    t = jnp.dot(t, W2, preferred_element_type=jnp.float32) + b2
    return _bn_fast(t, go, bo)


def _gin0_body(h_ref, p_ref, W1_ref, b1_ref, gm_ref, bm_ref, W2_ref, b2_ref,
               eps_ref, go_ref, bo_ref, out_ref):
    p = p_ref[...]
    agg = p[:N] + p[N_PAD:N_PAD + N]
    t = _gin_compute(h_ref[...], agg, W1_ref[...], b1_ref[...], gm_ref[...],
                     bm_ref[...], W2_ref[...], b2_ref[...], eps_ref[0, 0],
                     go_ref[...], bo_ref[...])
    out_ref[...] = jnp.maximum(t, 0.0)


def _gin1_body(h_ref, p_ref, W1_ref, b1_ref, gm_ref, bm_ref, W2_ref, b2_ref,
               eps_ref, go_ref, bo_ref, attW_ref, attb_ref, ab_ref):
    p = p_ref[...]
    agg = p[:N] + p[N_PAD:N_PAD + N]
    t = _gin_compute(h_ref[...], agg, W1_ref[...], b1_ref[...], gm_ref[...],
                     bm_ref[...], W2_ref[...], b2_ref[...], eps_ref[0, 0],
                     go_ref[...], bo_ref[...])
    attW = attW_ref[...]
    a = jnp.dot(t, attW[:F], preferred_element_type=jnp.float32) + attb_ref[0, 0]
    b = jnp.dot(t, attW[F:], preferred_element_type=jnp.float32)
    ab_ref[...] = jnp.concatenate([a, b], axis=1)


_gin0 = pl.pallas_call(
    _gin0_body,
    out_shape=jax.ShapeDtypeStruct((N, F), jnp.float32),
)

_gin1 = pl.pallas_call(
    _gin1_body,
    out_shape=jax.ShapeDtypeStruct((N, 2), jnp.float32),
)


def kernel(x, edge_index, p0_W1, p0_b1, p0_gm, p0_bm, p0_W2, p0_b2, p0_eps,
           p0_go, p0_bo, p1_W1, p1_b1, p1_gm, p1_bm, p1_W2, p1_b2, p1_eps,
           p1_go, p1_bo, att_W, att_b):
    src = edge_index[0]
    dst = edge_index[1]
    src3 = src.reshape(TILES, NSLAB, IB, CHUNK)
    dst3 = dst.reshape(TILES, NSLAB, IB, CHUNK)
    zeros = jnp.zeros((N_PAD, F), jnp.float32)

    def r2(v):
        return v.reshape(1, -1)

    part0 = _seg_sum(x, src3, dst3, zeros)
    h = _gin0(x, part0, p0_W1, r2(p0_b1), r2(p0_gm), r2(p0_bm), p0_W2,
              r2(p0_b2), r2(p0_eps), r2(p0_go), r2(p0_bo))
    part1 = _seg_sum(h, src3, dst3, zeros)
    ab = _gin1(h, part1, p1_W1, r2(p1_b1), r2(p1_gm), r2(p1_bm), p1_W2,
               r2(p1_b2), r2(p1_eps), r2(p1_go), r2(p1_bo), att_W, r2(att_b))
    return _score(ab[:, 0], ab[:, 1], src, dst)
